# baseline (speedup 1.0000x reference)
import functools

import jax
import jax.numpy as jnp
from jax import lax
from jax.experimental import pallas as pl
from jax.experimental.pallas import tpu as pltpu
from jax.experimental.pallas import tpu_sc as plsc

SLOPE = 0.01
LANES = 16
NUM_CORES = 2
NUM_SUBCORES = 16
ZROWS = 104
WIN = 2000
DRAIN = 256
CBUF = 304
NSEG = DRAIN // LANES
EIDB = 19


def _lrelu(t):
    return jnp.where(t >= 0, t, t * SLOPE)


def _edge_embed_body(ea_ref, w_ref, b_ref, out_ref):
    out_ref[...] = (
        jnp.dot(ea_ref[...], w_ref[...], preferred_element_type=jnp.float32)
        + b_ref[...]
    )


def _edge_embed(edge_attr, w, b):
    E, K = edge_attr.shape
    D = w.shape[1]
    BLK = 4000
    grid = (E // BLK,)
    return pl.pallas_call(
        _edge_embed_body,
        grid=grid,
        in_specs=[
            pl.BlockSpec((BLK, K), lambda i: (i, 0)),
            pl.BlockSpec((K, D), lambda i: (0, 0)),
            pl.BlockSpec((1, D), lambda i: (0, 0)),
        ],
        out_specs=pl.BlockSpec((BLK, D), lambda i: (i, 0)),
        out_shape=jax.ShapeDtypeStruct((E, D), jnp.float32),
    )(edge_attr, w, b.reshape(1, D))


def _sc_aggregate(h, e, src, dst):
    N, D = h.shape
    E = src.shape[0]
    half = N // NUM_CORES
    rpt = half // NUM_SUBCORES // 8 * 8
    tail = half - NUM_SUBCORES * rpt
    nvec = D // LANES
    nwin = E // WIN
    npair = nwin // 2
    groups = WIN // LANES

    mesh = plsc.VectorSubcoreMesh(core_axis_name="c", subcore_axis_name="s")

    @functools.partial(
        pl.kernel,
        mesh=mesh,
        out_type=jax.ShapeDtypeStruct((N, D), jnp.float32),
        scratch_types=[
            pltpu.VMEM((WIN,), jnp.int32),
            pltpu.VMEM((WIN,), jnp.int32),
            pltpu.VMEM((CBUF,), jnp.int32),
            pltpu.VMEM((DRAIN + LANES,), jnp.int32),
            pltpu.VMEM((DRAIN + LANES,), jnp.int32),
            pltpu.VMEM((NSEG + 1, LANES), jnp.int32),
            pltpu.VMEM((DRAIN, D), jnp.float32),
            pltpu.VMEM((DRAIN, D), jnp.float32),
            pltpu.VMEM((ZROWS, D), jnp.float32),
            pltpu.VMEM_SHARED((half, D), jnp.float32),
            pltpu.SemaphoreType.DMA,
            pltpu.SemaphoreType.DMA,
            pltpu.SemaphoreType.DMA,
            pltpu.SemaphoreType.DMA,
            pltpu.SemaphoreType.DMA,
        ],
    )
    def k(h_hbm, e_hbm, src_hbm, dst_hbm, out_hbm,
          dstw0_v, dstw1_v, pck_v, eidc_v, srcc_v, dst2_v,
          hrow_v, erow_v, z_v, acc_sh, sem0, sem1, semh, seme, semf):
        c = lax.axis_index("c")
        s = lax.axis_index("s")
        lo = c * half + s * rpt
        lo_loc = s * rpt
        is_last = s == NUM_SUBCORES - 1
        hi = jnp.where(is_last, lo + rpt + tail, lo + rpt)
        iota16 = lax.iota(jnp.int32, LANES)
        shift_idx = [jnp.maximum(iota16 - kk, 0) for kk in (1, 2, 4, 8)]
        sems = (sem0, sem1)
        dstws = (dstw0_v, dstw1_v)
        emask = (1 << EIDB) - 1

        def zrow(i, carry):
            for j in range(nvec):
                z_v[i, pl.ds(j * LANES, LANES)] = jnp.zeros((LANES,), jnp.float32)
            return carry
        lax.fori_loop(0, ZROWS, zrow, 0)
        for r in range(rpt // ZROWS):
            pltpu.sync_copy(z_v, acc_sh.at[pl.ds(lo_loc + r * ZROWS, ZROWS), :])
        if tail:
            @pl.when(is_last)
            def _():
                pltpu.sync_copy(z_v.at[pl.ds(0, tail), :],
                                acc_sh.at[pl.ds(lo_loc + rpt, tail), :])
        plsc.subcore_barrier()

        def unpack(nseg):
            def seg_body(g, carry):
                v16 = pck_v[pl.ds(g * LANES, LANES)]
                eidc_v[pl.ds(g * LANES, LANES)] = v16 & emask
                dst2_v[g, :] = (v16 >> EIDB) + lo_loc
                return carry
            lax.fori_loop(0, nseg, seg_body, 0)

        def fire_src(nseg):
            def seg_body(g, carry):
                sl = pl.ds(g * LANES, LANES)
                pltpu.async_copy(src_hbm.at[eidc_v.at[sl]], srcc_v.at[sl], semf)
                return carry
            lax.fori_loop(0, nseg, seg_body, 0)

        def wait_src(nseg):
            def seg_body(g, carry):
                sl = pl.ds(g * LANES, LANES)
                pltpu.make_async_copy(src_hbm.at[eidc_v.at[sl]],
                                      srcc_v.at[sl], semf).wait()
                return carry
            lax.fori_loop(0, nseg, seg_body, 0)

        def fire_rows(nseg):
            def seg_body(g, carry):
                sl = pl.ds(g * LANES, LANES)
                pltpu.async_copy(h_hbm.at[srcc_v.at[sl]], hrow_v.at[sl, :], semh)
                pltpu.async_copy(e_hbm.at[eidc_v.at[sl]], erow_v.at[sl, :], seme)
                return carry
            lax.fori_loop(0, nseg, seg_body, 0)

        def wait_rows(nseg):
            def seg_body(g, carry):
                sl = pl.ds(g * LANES, LANES)
                pltpu.make_async_copy(h_hbm.at[srcc_v.at[sl]],
                                      hrow_v.at[sl, :], semh).wait()
                pltpu.make_async_copy(e_hbm.at[eidc_v.at[sl]],
                                      erow_v.at[sl, :], seme).wait()
                return carry
            lax.fori_loop(0, nseg, seg_body, 0)

        def compute_msgs(nseg):
            def row_body(r, carry):
                for j in range(nvec):
                    sl = pl.ds(j * LANES, LANES)
                    hrow_v[r, sl] = jnp.maximum(hrow_v[r, sl] + erow_v[r, sl], 0.0)
                return carry
            lax.fori_loop(0, nseg * LANES, row_body, 0)

        def scatter_msgs(nseg):
            def seg_body(g, carry):
                pltpu.sync_copy(hrow_v.at[pl.ds(g * LANES, LANES), :],
                                acc_sh.at[dst2_v.at[g]], add=True)
                return carry
            lax.fori_loop(0, nseg, seg_body, 0)

        def drain(nseg):
            unpack(nseg)
            fire_src(nseg)
            wait_src(nseg)
            fire_rows(nseg)
            wait_rows(nseg)
            compute_msgs(nseg)
            scatter_msgs(nseg)

        def drain_full(cnt):
            drain(NSEG)
            remv = pck_v[pl.ds(DRAIN, LANES)]
            pck_v[pl.ds(0, LANES)] = remv
            return cnt - DRAIN

        def stage(wi, b):
            base = wi * WIN
            pltpu.async_copy(dst_hbm.at[pl.ds(base, WIN)], dstws[b], sems[b])

        def unstage(b):
            pltpu.make_async_copy(dst_hbm.at[pl.ds(0, WIN)],
                                  dstws[b], sems[b]).wait()

        for b in range(2):
            stage(b, b)

        def pair_body(p, cnt):
            for b in range(2):
                wi = p * 2 + b
                unstage(b)
                base = wi * WIN

                def group_body(g, cnt):
                    d16 = dstws[b][pl.ds(g * LANES, LANES)]
                    msk = (d16 >= lo) & (d16 < hi)
                    mi = jnp.where(msk, 1, 0)
                    t = mi
                    for sv in shift_idx:
                        t = jnp.maximum(t, t[sv])
                    anym = t[LANES - 1]
                    pv = ((d16 - lo) << EIDB) | (base + g * LANES + iota16)

                    def gated(cnt):
                        for kk in range(LANES):
                            mk = mi[kk]

                            @pl.when(mk == 1)
                            def _():
                                pck_v[pl.ds(cnt, LANES)] = lax.broadcast(
                                    pv[kk], (LANES,))
                            cnt = cnt + mk
                        return cnt
                    cnt = lax.cond(anym > 0, gated, lambda x: x, cnt)
                    return lax.cond(cnt >= DRAIN, drain_full, lambda x: x, cnt)
                cnt = lax.fori_loop(0, groups, group_body, cnt)

                @pl.when(p < npair - 1)
                def _():
                    stage(wi + 2, b)
            return cnt
        cnt = lax.fori_loop(0, npair, pair_body, 0)

        pck_v[pl.ds(cnt, LANES)] = jnp.zeros((LANES,), jnp.int32)
        npad = (LANES - cnt % LANES) % LANES
        nseg = (cnt + npad) // LANES
        unpack(nseg)
        fire_src(nseg)
        wait_src(nseg)
        fire_rows(nseg)
        wait_rows(nseg)
        compute_msgs(nseg)
        zf = jnp.zeros((LANES,), jnp.float32)
        for kk in range(LANES):
            row = (nseg - 1) * LANES + kk

            @pl.when((row >= cnt) & (row >= 0))
            def _():
                for j in range(nvec):
                    hrow_v[row, pl.ds(j * LANES, LANES)] = zf
        scatter_msgs(nseg)

        plsc.subcore_barrier()
        pltpu.sync_copy(acc_sh.at[pl.ds(lo_loc, rpt), :],
                        out_hbm.at[pl.ds(lo, rpt), :])
        if tail:
            @pl.when(is_last)
            def _():
                pltpu.sync_copy(acc_sh.at[pl.ds(lo_loc + rpt, tail), :],
                                out_hbm.at[pl.ds(lo + rpt, tail), :])

    return k(h, e, src, dst)


def _dense_body(nlrelu, h_ref, a_ref, w1_ref, b1_ref, g_ref, bt_ref,
                w2_ref, b2_ref, out_ref):
    x = h_ref[...] + a_ref[...]
    t = jnp.dot(x, w1_ref[...], preferred_element_type=jnp.float32) + b1_ref[...]
    mean = jnp.mean(t, axis=0, keepdims=True)
    var = jnp.mean((t - mean) ** 2, axis=0, keepdims=True)
    t = (t - mean) * lax.rsqrt(var + 1e-5) * g_ref[...] + bt_ref[...]
    t = _lrelu(t)
    t = jnp.dot(t, w2_ref[...], preferred_element_type=jnp.float32) + b2_ref[...]
    for _ in range(nlrelu):
        t = _lrelu(t)
    out_ref[...] = t


def _dense(h, agg, w1, b1, gamma, beta, w2, b2, nlrelu):
    N, D = h.shape
    H = w1.shape[1]
    return pl.pallas_call(
        functools.partial(_dense_body, nlrelu),
        out_shape=jax.ShapeDtypeStruct((N, H), jnp.float32),
    )(h, agg, w1, b1.reshape(1, H), gamma.reshape(1, H), beta.reshape(1, H),
      w2, b2.reshape(1, H))


def _head_body(h_ref, wr_ref, br_ref, we_ref, be_ref, out_ref):
    t = jnp.dot(h_ref[...], wr_ref[...], preferred_element_type=jnp.float32)
    t = _lrelu(t + br_ref[...])
    out_ref[...] = (
        jnp.dot(t, we_ref[...], preferred_element_type=jnp.float32) + be_ref[...]
    )


def _head(h, wr, br, wend, bend):
    N, D = h.shape
    R = wr.shape[1]
    BLK = 1000
    return pl.pallas_call(
        _head_body,
        grid=(N // BLK,),
        in_specs=[
            pl.BlockSpec((BLK, D), lambda i: (i, 0)),
            pl.BlockSpec((D, R), lambda i: (0, 0)),
            pl.BlockSpec((1, R), lambda i: (0, 0)),
            pl.BlockSpec((R, 1), lambda i: (0, 0)),
            pl.BlockSpec((1, 1), lambda i: (0, 0)),
        ],
        out_specs=pl.BlockSpec((BLK, 1), lambda i: (i, 0)),
        out_shape=jax.ShapeDtypeStruct((N, 1), jnp.float32),
    )(h, wr, br.reshape(1, R), wend, bend.reshape(1, 1))


def kernel(x, edge_index, edge_attr, batch, params):
    src = edge_index[0]
    dst = edge_index[1]
    h = x
    num_layers = 3
    for l in range(num_layers):
        e = _edge_embed(edge_attr, params['We_%d' % l], params['be_%d' % l])
        agg = _sc_aggregate(h, e, src, dst)
        h = _dense(h, agg,
                   params['W1_%d' % l], params['b1_%d' % l],
                   params['gamma_%d' % l], params['beta_%d' % l],
                   params['W2_%d' % l], params['b2_%d' % l],
                   nlrelu=2 if l < num_layers - 1 else 1)
    return _head(h, params['Wr'], params['br'], params['Wend'], params['bend'])


# speedup vs baseline: 1.0580x; 1.0580x over previous
import functools

import jax
import jax.numpy as jnp
from jax import lax
from jax.experimental import pallas as pl
from jax.experimental.pallas import tpu as pltpu
from jax.experimental.pallas import tpu_sc as plsc

SLOPE = 0.01
LANES = 16
NUM_CORES = 2
NUM_SUBCORES = 16
ZROWS = 78
WIN = 2000
DRAIN = 256
CBUF = 304
NSEG = DRAIN // LANES
EIDB = 19


def _lrelu(t):
    return jnp.where(t >= 0, t, t * SLOPE)


def _edge_embed_body(ea_ref, w_ref, b_ref, out_ref):
    out_ref[...] = (
        jnp.dot(ea_ref[...], w_ref[...], preferred_element_type=jnp.float32)
        + b_ref[...]
    )


def _edge_embed(edge_attr, w, b):
    E, K = edge_attr.shape
    D = w.shape[1]
    BLK = 4000
    grid = (E // BLK,)
    return pl.pallas_call(
        _edge_embed_body,
        grid=grid,
        in_specs=[
            pl.BlockSpec((BLK, K), lambda i: (i, 0)),
            pl.BlockSpec((K, D), lambda i: (0, 0)),
            pl.BlockSpec((1, D), lambda i: (0, 0)),
        ],
        out_specs=pl.BlockSpec((BLK, D), lambda i: (i, 0)),
        out_shape=jax.ShapeDtypeStruct((E, D), jnp.float32),
    )(edge_attr, w, b.reshape(1, D))


def _sc_aggregate(h, e, src, dst):
    N, D = h.shape
    E = src.shape[0]
    half = N // NUM_CORES
    rpt = half // NUM_SUBCORES // 8 * 8
    tail = half - NUM_SUBCORES * rpt
    nvec = D // LANES
    nwin = E // WIN
    npair = nwin // 2
    groups = WIN // LANES

    mesh = plsc.VectorSubcoreMesh(core_axis_name="c", subcore_axis_name="s")

    @functools.partial(
        pl.kernel,
        mesh=mesh,
        out_type=jax.ShapeDtypeStruct((N, D), jnp.float32),
        scratch_types=[
            pltpu.VMEM((WIN,), jnp.int32),
            pltpu.VMEM((WIN,), jnp.int32),
            pltpu.VMEM((WIN,), jnp.int32),
            pltpu.VMEM((WIN,), jnp.int32),
            pltpu.VMEM((CBUF,), jnp.int32),
            pltpu.VMEM((CBUF,), jnp.int32),
            pltpu.VMEM((2, 128), jnp.int32),
            pltpu.VMEM((DRAIN + LANES,), jnp.int32),
            pltpu.VMEM((NSEG + 1, LANES), jnp.int32),
            pltpu.VMEM((DRAIN, D), jnp.float32),
            pltpu.VMEM((DRAIN, D), jnp.float32),
            pltpu.VMEM((ZROWS, D), jnp.float32),
            pltpu.VMEM_SHARED((half, D), jnp.float32),
            pltpu.SemaphoreType.DMA,
            pltpu.SemaphoreType.DMA,
            pltpu.SemaphoreType.DMA,
            pltpu.SemaphoreType.DMA,
            pltpu.SemaphoreType.DMA,
        ],
    )
    def k(h_hbm, e_hbm, src_hbm, dst_hbm, out_hbm,
          dstw0_v, dstw1_v, srcw0_v, srcw1_v, pck_v, psrc_v, dst2r_v,
          eidc_v, dst2_v,
          hrow_v, erow_v, z_v, acc_sh, sem0, sem1, semh, seme, semf):
        c = lax.axis_index("c")
        s = lax.axis_index("s")
        lo = c * half + s * rpt
        lo_loc = s * rpt
        is_last = s == NUM_SUBCORES - 1
        hi = jnp.where(is_last, lo + rpt + tail, lo + rpt)
        iota16 = lax.iota(jnp.int32, LANES)
        shift_idx = [jnp.maximum(iota16 - kk, 0) for kk in (1, 2, 4, 8)]
        sems = (sem0, sem1)
        dstws = (dstw0_v, dstw1_v)
        srcws = (srcw0_v, srcw1_v)
        emask = (1 << EIDB) - 1

        def zrow(i, carry):
            for j in range(nvec):
                z_v[i, pl.ds(j * LANES, LANES)] = jnp.zeros((LANES,), jnp.float32)
            return carry
        lax.fori_loop(0, ZROWS, zrow, 0)
        for r in range(rpt // ZROWS):
            pltpu.sync_copy(z_v, acc_sh.at[pl.ds(lo_loc + r * ZROWS, ZROWS), :])
        if tail:
            @pl.when(is_last)
            def _():
                pltpu.sync_copy(z_v.at[pl.ds(0, tail), :],
                                acc_sh.at[pl.ds(lo_loc + rpt, tail), :])
        plsc.subcore_barrier()

        def unpack(nseg):
            def seg_body(g, carry):
                v16 = pck_v[pl.ds(g * LANES, LANES)]
                eidc_v[pl.ds(g * LANES, LANES)] = v16 & emask
                rows = (v16 >> EIDB) + lo_loc
                dst2_v[g, :] = rows
                dst2r_v[g // 8, pl.ds((g % 8) * LANES, LANES)] = rows
                return carry
            lax.fori_loop(0, nseg, seg_body, 0)

        def fire_rows(nseg):
            def seg_body(g, carry):
                sl = pl.ds(g * LANES, LANES)
                pltpu.async_copy(h_hbm.at[psrc_v.at[sl]], hrow_v.at[sl, :], semh)
                pltpu.async_copy(e_hbm.at[eidc_v.at[sl]], erow_v.at[sl, :], seme)
                return carry
            lax.fori_loop(0, nseg, seg_body, 0)

        def wait_rows(nseg):
            def seg_body(g, carry):
                sl = pl.ds(g * LANES, LANES)
                pltpu.make_async_copy(h_hbm.at[psrc_v.at[sl]],
                                      hrow_v.at[sl, :], semh).wait()
                pltpu.make_async_copy(e_hbm.at[eidc_v.at[sl]],
                                      erow_v.at[sl, :], seme).wait()
                return carry
            lax.fori_loop(0, nseg, seg_body, 0)

        def compute_msgs(nseg):
            def row_body(r, carry):
                for j in range(nvec):
                    sl = pl.ds(j * LANES, LANES)
                    hrow_v[r, sl] = jnp.maximum(hrow_v[r, sl] + erow_v[r, sl], 0.0)
                return carry
            lax.fori_loop(0, nseg * LANES, row_body, 0)

        def scatter_msgs(nseg):
            def seg_body(g, carry):
                pltpu.sync_copy(hrow_v.at[pl.ds(g * LANES, LANES), :],
                                acc_sh.at[dst2_v.at[g]], add=True)
                return carry
            lax.fori_loop(0, nseg, seg_body, 0)

        def drain_full(cnt):
            unpack(NSEG)
            fire_rows(NSEG)
            wait_rows(NSEG)
            compute_msgs(NSEG)
            pltpu.sync_copy(hrow_v.at[pl.ds(0, 128), :],
                            acc_sh.at[dst2r_v.at[0]], add=True)
            pltpu.sync_copy(hrow_v.at[pl.ds(128, 128), :],
                            acc_sh.at[dst2r_v.at[1]], add=True)
            remv = pck_v[pl.ds(DRAIN, LANES)]
            pck_v[pl.ds(0, LANES)] = remv
            rems = psrc_v[pl.ds(DRAIN, LANES)]
            psrc_v[pl.ds(0, LANES)] = rems
            return cnt - DRAIN

        def stage(wi, b):
            base = wi * WIN
            pltpu.async_copy(dst_hbm.at[pl.ds(base, WIN)], dstws[b], sems[b])
            pltpu.async_copy(src_hbm.at[pl.ds(base, WIN)], srcws[b], sems[b])

        def unstage(b):
            pltpu.make_async_copy(dst_hbm.at[pl.ds(0, WIN)],
                                  dstws[b], sems[b]).wait()
            pltpu.make_async_copy(src_hbm.at[pl.ds(0, WIN)],
                                  srcws[b], sems[b]).wait()

        for b in range(2):
            stage(b, b)

        def pair_body(p, cnt):
            for b in range(2):
                wi = p * 2 + b
                unstage(b)
                base = wi * WIN

                def group_body(g, cnt):
                    d16 = dstws[b][pl.ds(g * LANES, LANES)]
                    msk = (d16 >= lo) & (d16 < hi)
                    mi = jnp.where(msk, 1, 0)
                    t = mi << iota16
                    for kk, sv in zip((1, 2, 4, 8), shift_idx):
                        t = t + jnp.where(iota16 >= kk, t[sv], 0)
                    bits = t[LANES - 1]

                    def gated(cnt):
                        s16 = srcws[b][pl.ds(g * LANES, LANES)]
                        pv = ((d16 - lo) << EIDB) | (base + g * LANES + iota16)
                        for kk in range(LANES):
                            mk = (bits >> kk) & 1

                            @pl.when(mk == 1)
                            def _():
                                pck_v[pl.ds(cnt, LANES)] = lax.broadcast(
                                    pv[kk], (LANES,))
                                psrc_v[pl.ds(cnt, LANES)] = lax.broadcast(
                                    s16[kk], (LANES,))
                            cnt = cnt + mk
                        return cnt
                    cnt = lax.cond(bits > 0, gated, lambda x: x, cnt)
                    return lax.cond(cnt >= DRAIN, drain_full, lambda x: x, cnt)
                cnt = lax.fori_loop(0, groups, group_body, cnt)

                @pl.when(p < npair - 1)
                def _():
                    stage(wi + 2, b)
            return cnt
        cnt = lax.fori_loop(0, npair, pair_body, 0)

        pck_v[pl.ds(cnt, LANES)] = jnp.zeros((LANES,), jnp.int32)
        psrc_v[pl.ds(cnt, LANES)] = jnp.zeros((LANES,), jnp.int32)
        npad = (LANES - cnt % LANES) % LANES
        nseg = (cnt + npad) // LANES
        unpack(nseg)
        fire_rows(nseg)
        wait_rows(nseg)
        compute_msgs(nseg)
        zf = jnp.zeros((LANES,), jnp.float32)
        for kk in range(LANES):
            row = (nseg - 1) * LANES + kk

            @pl.when((row >= cnt) & (row >= 0))
            def _():
                for j in range(nvec):
                    hrow_v[row, pl.ds(j * LANES, LANES)] = zf
        scatter_msgs(nseg)

        plsc.subcore_barrier()
        pltpu.sync_copy(acc_sh.at[pl.ds(lo_loc, rpt), :],
                        out_hbm.at[pl.ds(lo, rpt), :])
        if tail:
            @pl.when(is_last)
            def _():
                pltpu.sync_copy(acc_sh.at[pl.ds(lo_loc + rpt, tail), :],
                                out_hbm.at[pl.ds(lo + rpt, tail), :])

    return k(h, e, src, dst)


def _dense_body(nlrelu, h_ref, a_ref, w1_ref, b1_ref, g_ref, bt_ref,
                w2_ref, b2_ref, out_ref):
    x = h_ref[...] + a_ref[...]
    t = jnp.dot(x, w1_ref[...], preferred_element_type=jnp.float32) + b1_ref[...]
    mean = jnp.mean(t, axis=0, keepdims=True)
    var = jnp.mean((t - mean) ** 2, axis=0, keepdims=True)
    t = (t - mean) * lax.rsqrt(var + 1e-5) * g_ref[...] + bt_ref[...]
    t = _lrelu(t)
    t = jnp.dot(t, w2_ref[...], preferred_element_type=jnp.float32) + b2_ref[...]
    for _ in range(nlrelu):
        t = _lrelu(t)
    out_ref[...] = t


def _dense(h, agg, w1, b1, gamma, beta, w2, b2, nlrelu):
    N, D = h.shape
    H = w1.shape[1]
    return pl.pallas_call(
        functools.partial(_dense_body, nlrelu),
        out_shape=jax.ShapeDtypeStruct((N, H), jnp.float32),
    )(h, agg, w1, b1.reshape(1, H), gamma.reshape(1, H), beta.reshape(1, H),
      w2, b2.reshape(1, H))


def _head_body(h_ref, wr_ref, br_ref, we_ref, be_ref, out_ref):
    t = jnp.dot(h_ref[...], wr_ref[...], preferred_element_type=jnp.float32)
    t = _lrelu(t + br_ref[...])
    out_ref[...] = (
        jnp.dot(t, we_ref[...], preferred_element_type=jnp.float32) + be_ref[...]
    )


def _head(h, wr, br, wend, bend):
    N, D = h.shape
    R = wr.shape[1]
    BLK = 1000
    return pl.pallas_call(
        _head_body,
        grid=(N // BLK,),
        in_specs=[
            pl.BlockSpec((BLK, D), lambda i: (i, 0)),
            pl.BlockSpec((D, R), lambda i: (0, 0)),
            pl.BlockSpec((1, R), lambda i: (0, 0)),
            pl.BlockSpec((R, 1), lambda i: (0, 0)),
            pl.BlockSpec((1, 1), lambda i: (0, 0)),
        ],
        out_specs=pl.BlockSpec((BLK, 1), lambda i: (i, 0)),
        out_shape=jax.ShapeDtypeStruct((N, 1), jnp.float32),
    )(h, wr, br.reshape(1, R), wend, bend.reshape(1, 1))


def kernel(x, edge_index, edge_attr, batch, params):
    src = edge_index[0]
    dst = edge_index[1]
    h = x
    num_layers = 3
    for l in range(num_layers):
        e = _edge_embed(edge_attr, params['We_%d' % l], params['be_%d' % l])
        agg = _sc_aggregate(h, e, src, dst)
        h = _dense(h, agg,
                   params['W1_%d' % l], params['b1_%d' % l],
                   params['gamma_%d' % l], params['beta_%d' % l],
                   params['W2_%d' % l], params['b2_%d' % l],
                   nlrelu=2 if l < num_layers - 1 else 1)
    return _head(h, params['Wr'], params['br'], params['Wend'], params['bend'])


# speedup vs baseline: 1.6354x; 1.5457x over previous
import functools

import jax
import jax.numpy as jnp
from jax import lax
from jax.experimental import pallas as pl
from jax.experimental.pallas import tpu as pltpu
from jax.experimental.pallas import tpu_sc as plsc

SLOPE = 0.01
LANES = 16
NUM_CORES = 2
NUM_SUBCORES = 16
ZROWS = 78
WIN = 2000
DRAIN = 256
CBUF = 368
NSEG = DRAIN // LANES
EIDB = 19


def _lrelu(t):
    return jnp.where(t >= 0, t, t * SLOPE)


def _edge_embed_body(ea_ref, w_ref, b_ref, out_ref):
    out_ref[...] = (
        jnp.dot(ea_ref[...], w_ref[...], preferred_element_type=jnp.float32)
        + b_ref[...]
    )


def _edge_embed(edge_attr, w, b):
    E, K = edge_attr.shape
    D = w.shape[1]
    BLK = 4000
    grid = (E // BLK,)
    return pl.pallas_call(
        _edge_embed_body,
        grid=grid,
        in_specs=[
            pl.BlockSpec((BLK, K), lambda i: (i, 0)),
            pl.BlockSpec((K, D), lambda i: (0, 0)),
            pl.BlockSpec((1, D), lambda i: (0, 0)),
        ],
        out_specs=pl.BlockSpec((BLK, D), lambda i: (i, 0)),
        out_shape=jax.ShapeDtypeStruct((E, D), jnp.float32),
    )(edge_attr, w, b.reshape(1, D))


def _sc_aggregate(h, e, src, dst):
    N, D = h.shape
    E = src.shape[0]
    half = N // NUM_CORES
    rpt = half // NUM_SUBCORES // 8 * 8
    tail = half - NUM_SUBCORES * rpt
    nvec = D // LANES
    nwin = E // WIN
    npair = nwin // 2
    groups = WIN // LANES

    mesh = plsc.VectorSubcoreMesh(core_axis_name="c", subcore_axis_name="s")

    @functools.partial(
        pl.kernel,
        mesh=mesh,
        out_type=jax.ShapeDtypeStruct((N, D), jnp.float32),
        scratch_types=[
            pltpu.VMEM((WIN,), jnp.int32),
            pltpu.VMEM((WIN,), jnp.int32),
            pltpu.VMEM((WIN,), jnp.int32),
            pltpu.VMEM((WIN,), jnp.int32),
            pltpu.VMEM((CBUF,), jnp.int32),
            pltpu.VMEM((CBUF,), jnp.int32),
            pltpu.VMEM((2, 128), jnp.int32),
            pltpu.VMEM((DRAIN + LANES,), jnp.int32),
            pltpu.VMEM((NSEG + 1, LANES), jnp.int32),
            pltpu.VMEM((DRAIN, D), jnp.float32),
            pltpu.VMEM((DRAIN, D), jnp.float32),
            pltpu.VMEM((ZROWS, D), jnp.float32),
            pltpu.VMEM_SHARED((half, D), jnp.float32),
            pltpu.SemaphoreType.DMA,
            pltpu.SemaphoreType.DMA,
            pltpu.SemaphoreType.DMA,
            pltpu.SemaphoreType.DMA,
            pltpu.SemaphoreType.DMA,
        ],
    )
    def k(h_hbm, e_hbm, src_hbm, dst_hbm, out_hbm,
          dstw0_v, dstw1_v, srcw0_v, srcw1_v, pck_v, psrc_v, dst2r_v,
          eidc_v, dst2_v,
          hrow_v, erow_v, z_v, acc_sh, sem0, sem1, semh, seme, semf):
        c = lax.axis_index("c")
        s = lax.axis_index("s")
        lo = c * half + s * rpt
        lo_loc = s * rpt
        is_last = s == NUM_SUBCORES - 1
        hi = jnp.where(is_last, lo + rpt + tail, lo + rpt)
        iota16 = lax.iota(jnp.int32, LANES)
        shift_idx = [jnp.maximum(iota16 - kk, 0) for kk in (1, 2, 4, 8)]
        sems = (sem0, sem1)
        dstws = (dstw0_v, dstw1_v)
        srcws = (srcw0_v, srcw1_v)
        emask = (1 << EIDB) - 1

        def zrow(i, carry):
            for j in range(nvec):
                z_v[i, pl.ds(j * LANES, LANES)] = jnp.zeros((LANES,), jnp.float32)
            return carry
        lax.fori_loop(0, ZROWS, zrow, 0)
        for r in range(rpt // ZROWS):
            pltpu.sync_copy(z_v, acc_sh.at[pl.ds(lo_loc + r * ZROWS, ZROWS), :])
        if tail:
            @pl.when(is_last)
            def _():
                pltpu.sync_copy(z_v.at[pl.ds(0, tail), :],
                                acc_sh.at[pl.ds(lo_loc + rpt, tail), :])
        plsc.subcore_barrier()

        def unpack(nseg):
            def seg_body(g, carry):
                v16 = pck_v[pl.ds(g * LANES, LANES)]
                eidc_v[pl.ds(g * LANES, LANES)] = v16 & emask
                rows = (v16 >> EIDB) + lo_loc
                dst2_v[g, :] = rows
                dst2r_v[g // 8, pl.ds((g % 8) * LANES, LANES)] = rows
                return carry
            lax.fori_loop(0, nseg, seg_body, 0)

        def fire_rows(nseg):
            def seg_body(g, carry):
                sl = pl.ds(g * LANES, LANES)
                pltpu.async_copy(h_hbm.at[psrc_v.at[sl]], hrow_v.at[sl, :], semh)
                pltpu.async_copy(e_hbm.at[eidc_v.at[sl]], erow_v.at[sl, :], seme)
                return carry
            lax.fori_loop(0, nseg, seg_body, 0)

        def wait_rows(nseg):
            def seg_body(g, carry):
                sl = pl.ds(g * LANES, LANES)
                pltpu.make_async_copy(h_hbm.at[psrc_v.at[sl]],
                                      hrow_v.at[sl, :], semh).wait()
                pltpu.make_async_copy(e_hbm.at[eidc_v.at[sl]],
                                      erow_v.at[sl, :], seme).wait()
                return carry
            lax.fori_loop(0, nseg, seg_body, 0)

        def compute_msgs(nseg):
            def row_body(r, carry):
                for j in range(nvec):
                    sl = pl.ds(j * LANES, LANES)
                    hrow_v[r, sl] = jnp.maximum(hrow_v[r, sl] + erow_v[r, sl], 0.0)
                return carry
            lax.fori_loop(0, nseg * LANES, row_body, 0)

        def scatter_msgs(nseg):
            def seg_body(g, carry):
                pltpu.sync_copy(hrow_v.at[pl.ds(g * LANES, LANES), :],
                                acc_sh.at[dst2_v.at[g]], add=True)
                return carry
            lax.fori_loop(0, nseg, seg_body, 0)

        def drain_full(cnt):
            unpack(NSEG)
            fire_rows(NSEG)
            wait_rows(NSEG)
            compute_msgs(NSEG)
            pltpu.sync_copy(hrow_v.at[pl.ds(0, 128), :],
                            acc_sh.at[dst2r_v.at[0]], add=True)
            pltpu.sync_copy(hrow_v.at[pl.ds(128, 128), :],
                            acc_sh.at[dst2r_v.at[1]], add=True)
            remv = pck_v[pl.ds(DRAIN, LANES)]
            pck_v[pl.ds(0, LANES)] = remv
            rems = psrc_v[pl.ds(DRAIN, LANES)]
            psrc_v[pl.ds(0, LANES)] = rems
            return cnt - DRAIN

        def stage(wi, b):
            base = wi * WIN
            pltpu.async_copy(dst_hbm.at[pl.ds(base, WIN)], dstws[b], sems[b])
            pltpu.async_copy(src_hbm.at[pl.ds(base, WIN)], srcws[b], sems[b])

        def unstage(b):
            pltpu.make_async_copy(dst_hbm.at[pl.ds(0, WIN)],
                                  dstws[b], sems[b]).wait()
            pltpu.make_async_copy(src_hbm.at[pl.ds(0, WIN)],
                                  srcws[b], sems[b]).wait()

        for b in range(2):
            stage(b, b)

        def pair_body(p, cnt):
            for b in range(2):
                wi = p * 2 + b
                unstage(b)
                base = wi * WIN

                UNROLL = 5

                def block_body(blk, cnt):
                    infos = []
                    for u in range(UNROLL):
                        g = blk * UNROLL + u
                        d16 = dstws[b][pl.ds(g * LANES, LANES)]
                        msk = (d16 >= lo) & (d16 < hi)
                        mi = jnp.where(msk, 1, 0)
                        t = mi << iota16
                        for kk, sv in zip((1, 2, 4, 8), shift_idx):
                            t = t + jnp.where(iota16 >= kk, t[sv], 0)
                        infos.append((t[LANES - 1], d16, g))

                    for bits, d16, g in infos:
                        def gated(cnt, bits=bits, d16=d16, g=g):
                            s16 = srcws[b][pl.ds(g * LANES, LANES)]
                            pv = ((d16 - lo) << EIDB) | (base + g * LANES + iota16)
                            for kk in range(LANES):
                                mk = (bits >> kk) & 1

                                @pl.when(mk == 1)
                                def _():
                                    pck_v[pl.ds(cnt, LANES)] = lax.broadcast(
                                        pv[kk], (LANES,))
                                    psrc_v[pl.ds(cnt, LANES)] = lax.broadcast(
                                        s16[kk], (LANES,))
                                cnt = cnt + mk
                            return cnt
                        cnt = lax.cond(bits > 0, gated, lambda x: x, cnt)
                    return lax.cond(cnt >= DRAIN, drain_full, lambda x: x, cnt)
                cnt = lax.fori_loop(0, groups // UNROLL, block_body, cnt)

                @pl.when(p < npair - 1)
                def _():
                    stage(wi + 2, b)
            return cnt
        cnt = lax.fori_loop(0, npair, pair_body, 0)

        pck_v[pl.ds(cnt, LANES)] = jnp.zeros((LANES,), jnp.int32)
        psrc_v[pl.ds(cnt, LANES)] = jnp.zeros((LANES,), jnp.int32)
        npad = (LANES - cnt % LANES) % LANES
        nseg = (cnt + npad) // LANES
        unpack(nseg)
        fire_rows(nseg)
        wait_rows(nseg)
        compute_msgs(nseg)
        zf = jnp.zeros((LANES,), jnp.float32)
        for kk in range(LANES):
            row = (nseg - 1) * LANES + kk

            @pl.when((row >= cnt) & (row >= 0))
            def _():
                for j in range(nvec):
                    hrow_v[row, pl.ds(j * LANES, LANES)] = zf
        scatter_msgs(nseg)

        plsc.subcore_barrier()
        pltpu.sync_copy(acc_sh.at[pl.ds(lo_loc, rpt), :],
                        out_hbm.at[pl.ds(lo, rpt), :])
        if tail:
            @pl.when(is_last)
            def _():
                pltpu.sync_copy(acc_sh.at[pl.ds(lo_loc + rpt, tail), :],
                                out_hbm.at[pl.ds(lo + rpt, tail), :])

    return k(h, e, src, dst)


def _dense_body(nlrelu, h_ref, a_ref, w1_ref, b1_ref, g_ref, bt_ref,
                w2_ref, b2_ref, out_ref):
    x = h_ref[...] + a_ref[...]
    t = jnp.dot(x, w1_ref[...], preferred_element_type=jnp.float32) + b1_ref[...]
    mean = jnp.mean(t, axis=0, keepdims=True)
    var = jnp.mean((t - mean) ** 2, axis=0, keepdims=True)
    t = (t - mean) * lax.rsqrt(var + 1e-5) * g_ref[...] + bt_ref[...]
    t = _lrelu(t)
    t = jnp.dot(t, w2_ref[...], preferred_element_type=jnp.float32) + b2_ref[...]
    for _ in range(nlrelu):
        t = _lrelu(t)
    out_ref[...] = t


def _dense(h, agg, w1, b1, gamma, beta, w2, b2, nlrelu):
    N, D = h.shape
    H = w1.shape[1]
    return pl.pallas_call(
        functools.partial(_dense_body, nlrelu),
        out_shape=jax.ShapeDtypeStruct((N, H), jnp.float32),
    )(h, agg, w1, b1.reshape(1, H), gamma.reshape(1, H), beta.reshape(1, H),
      w2, b2.reshape(1, H))


def _head_body(h_ref, wr_ref, br_ref, we_ref, be_ref, out_ref):
    t = jnp.dot(h_ref[...], wr_ref[...], preferred_element_type=jnp.float32)
    t = _lrelu(t + br_ref[...])
    out_ref[...] = (
        jnp.dot(t, we_ref[...], preferred_element_type=jnp.float32) + be_ref[...]
    )


def _head(h, wr, br, wend, bend):
    N, D = h.shape
    R = wr.shape[1]
    BLK = 1000
    return pl.pallas_call(
        _head_body,
        grid=(N // BLK,),
        in_specs=[
            pl.BlockSpec((BLK, D), lambda i: (i, 0)),
            pl.BlockSpec((D, R), lambda i: (0, 0)),
            pl.BlockSpec((1, R), lambda i: (0, 0)),
            pl.BlockSpec((R, 1), lambda i: (0, 0)),
            pl.BlockSpec((1, 1), lambda i: (0, 0)),
        ],
        out_specs=pl.BlockSpec((BLK, 1), lambda i: (i, 0)),
        out_shape=jax.ShapeDtypeStruct((N, 1), jnp.float32),
    )(h, wr, br.reshape(1, R), wend, bend.reshape(1, 1))


def kernel(x, edge_index, edge_attr, batch, params):
    src = edge_index[0]
    dst = edge_index[1]
    h = x
    num_layers = 3
    for l in range(num_layers):
        e = _edge_embed(edge_attr, params['We_%d' % l], params['be_%d' % l])
        agg = _sc_aggregate(h, e, src, dst)
        h = _dense(h, agg,
                   params['W1_%d' % l], params['b1_%d' % l],
                   params['gamma_%d' % l], params['beta_%d' % l],
                   params['W2_%d' % l], params['b2_%d' % l],
                   nlrelu=2 if l < num_layers - 1 else 1)
    return _head(h, params['Wr'], params['br'], params['Wend'], params['bend'])
